# baseline (speedup 1.0000x reference)
import functools

import jax
import jax.numpy as jnp
from jax import lax
from jax.experimental import pallas as pl
from jax.experimental.pallas import tpu as pltpu
from jax.experimental.pallas import tpu_sc as plsc

SEQ = 2048
GAZ_NUM = 20
NLAYER = 4
WORD_DIM = 128
GAZ_DIM = 64
PAIR_DIM = 2 * GAZ_DIM
OUT_DIM = WORD_DIM + NLAYER * GAZ_DIM
RPP = NLAYER * GAZ_NUM
NW = 32
PPW = SEQ // NW
C = 2
RPC = C * RPP
CPW = PPW // C
OB = 8


def _sc_body(lg_hbm, cnt_hbm, msk_hbm, words_hbm, wtab_hbm, gtab_hbm,
             out_hbm, idxw, offw, cntw, mskw, ww, widx, wrows, rows, outv,
             sem_a, sem_b, sem_w, sem_s, sem_o):
    wid = lax.axis_index("s") * 2 + lax.axis_index("c")

    pltpu.sync_copy(lg_hbm.at[wid], idxw)
    scp = [pltpu.async_copy(cnt_hbm.at[wid], cntw, sem_s),
           pltpu.async_copy(msk_hbm.at[wid], mskw, sem_s),
           pltpu.async_copy(words_hbm.at[wid], widx, sem_s)]

    def xform_body(q, carry):
        for k in range(RPP // 16):
            v = idxw[q, pl.ds(k * 16, 16)]
            idxw[q, pl.ds(k * 16, 16)] = jnp.right_shift(v, 1)
            offw[q, pl.ds(k * 16, 16)] = (v & 1) * GAZ_DIM
        return carry

    lax.fori_loop(0, PPW, xform_body, 0)

    sems = (sem_a, sem_b)

    def fire(c, par):
        s = sems[par]
        base = par * RPC
        return [
            pltpu.async_copy(gtab_hbm.at[idxw.at[c * C + p]],
                             rows.at[pl.ds(base + p * RPP, RPP)], s)
            for p in range(C)
        ]

    def wait_rows(c, par):
        s = sems[par]
        base = par * RPC
        for p in range(C):
            pltpu.make_async_copy(gtab_hbm.at[idxw.at[c * C + p]],
                                  rows.at[pl.ds(base + p * RPP, RPP)],
                                  s).wait()

    fire(0, 0)
    for cp in scp:
        cp.wait()
    wcp = pltpu.async_copy(wtab_hbm.at[widx], wrows, sem_w)

    lanes = lax.iota(jnp.int32, 16)

    def weight_body(q, carry):
        cs = [cntw[q, pl.ds(k * 16, 16)] for k in range(5)]
        s = cs[0] + cs[1] + cs[2] + cs[3] + cs[4]
        for sh in (1, 2, 4, 8):
            s = s + s.at[jnp.bitwise_xor(lanes, sh)].get(
                mode="promise_in_bounds")
        rs = 4.0 / s
        for k in range(5):
            m = mskw[q, pl.ds(k * 16, 16)]
            ww[q, pl.ds(k * 16, 16)] = cs[k] * rs * (1.0 - m)
        return carry

    lax.fori_loop(0, PPW, weight_body, 0)
    wcp.wait()

    def out_row0(c_first):
        return pl.multiple_of(wid * PPW + c_first * C, OB)

    def compute_chunk(c, par, ob):
        for p in range(C):
            qg = c * C + p
            po = (qg % OB)

            def layer_body(l, carry2, _p=p, _qg=qg, _po=po):
                b = l * GAZ_NUM
                wv0 = ww[_qg, pl.ds(b, 16)]
                wv1 = ww[_qg, pl.ds(b + 4, 16)]
                ov0 = offw[_qg, pl.ds(b, 16)]
                ov1 = offw[_qg, pl.ds(b + 4, 16)]
                rbase = par * RPC + _p * RPP + b
                acc = [jnp.zeros((16,), jnp.float32) for _ in range(4)]
                for g in range(GAZ_NUM):
                    ws = wv0[g] if g < 16 else wv1[g - 4]
                    h = ov0[g] if g < 16 else ov1[g - 4]
                    for v in range(4):
                        acc[v] = acc[v] + ws * rows[rbase + g,
                                                    pl.ds(h + v * 16, 16)]
                for v in range(4):
                    outv[ob, _po, pl.ds(WORD_DIM + l * 64 + v * 16, 16)] = \
                        acc[v]
                return carry2

            lax.fori_loop(0, NLAYER, layer_body, 0)
            for v in range(WORD_DIM // 16):
                outv[ob, po, pl.ds(v * 16, 16)] = wrows[qg, pl.ds(v * 16, 16)]

    def pair_body(t, carry):
        for b in range(2):
            c = 2 * t + b
            ob = (c // 4) % 2
            if b == 0:
                fire(c + 1, 1 - b)
            else:
                @pl.when(t < CPW // 2 - 1)
                def _():
                    fire(c + 1, 1 - b)
            wait_rows(c, b)
            if b == 0:
                @pl.when((t % 2 == 0) & (t >= 4))
                def _():
                    pltpu.make_async_copy(
                        outv.at[ob], out_hbm.at[pl.ds(out_row0(c - 8), OB)],
                        sem_o).wait()
            compute_chunk(c, b, ob)
            if b == 1:
                @pl.when(t % 2 == 1)
                def _():
                    pltpu.async_copy(
                        outv.at[ob], out_hbm.at[pl.ds(out_row0(c - 3), OB)],
                        sem_o)
        return carry

    lax.fori_loop(0, CPW // 2, pair_body, 0)
    for ob in range(2):
        pltpu.make_async_copy(
            outv.at[ob], out_hbm.at[pl.ds(wid * PPW, OB)], sem_o).wait()


@jax.jit
def kernel(words, layer_gazs, gaz_count, gaz_mask, word_table, gaz_table):
    lg = layer_gazs.reshape(NW, PPW, RPP).astype(jnp.int32)
    cnt = gaz_count.reshape(NW, PPW, RPP)
    msk = gaz_mask.reshape(NW, PPW, RPP).astype(jnp.float32)
    wds = words.reshape(NW, PPW).astype(jnp.int32)
    gtab = gaz_table.reshape(gaz_table.shape[0] // 2, PAIR_DIM)

    mesh = plsc.VectorSubcoreMesh(core_axis_name="c", subcore_axis_name="s")
    f = functools.partial(
        pl.kernel,
        out_type=jax.ShapeDtypeStruct((SEQ, OUT_DIM), jnp.float32),
        mesh=mesh,
        compiler_params=pltpu.CompilerParams(use_tc_tiling_on_sc=True),
        scratch_types=[
            pltpu.VMEM((PPW, RPP), jnp.int32),
            pltpu.VMEM((PPW, RPP), jnp.int32),
            pltpu.VMEM((PPW, RPP), jnp.float32),
            pltpu.VMEM((PPW, RPP), jnp.float32),
            pltpu.VMEM((PPW, RPP), jnp.float32),
            pltpu.VMEM((PPW,), jnp.int32),
            pltpu.VMEM((PPW, WORD_DIM), jnp.float32),
            pltpu.VMEM((2 * RPC, PAIR_DIM), jnp.float32),
            pltpu.VMEM((2, OB, OUT_DIM), jnp.float32),
            pltpu.SemaphoreType.DMA,
            pltpu.SemaphoreType.DMA,
            pltpu.SemaphoreType.DMA,
            pltpu.SemaphoreType.DMA,
            pltpu.SemaphoreType.DMA,
        ],
    )(_sc_body)
    return f(lg, cnt, msk, wds, word_table, gtab)


# speedup vs baseline: 1.1229x; 1.1229x over previous
import functools

import jax
import jax.numpy as jnp
from jax import lax
from jax.experimental import pallas as pl
from jax.experimental.pallas import tpu as pltpu
from jax.experimental.pallas import tpu_sc as plsc

SEQ = 2048
GAZ_NUM = 20
NLAYER = 4
WORD_DIM = 128
GAZ_DIM = 64
PAIR_DIM = 2 * GAZ_DIM
OUT_DIM = WORD_DIM + NLAYER * GAZ_DIM
RPP = NLAYER * GAZ_NUM
NW = 32
PPW = SEQ // NW
C = 2
RPC = C * RPP
CPW = PPW // C
OB = 8


def _sc_body(lg_hbm, cnt_hbm, msk_hbm, words_hbm, wtab_hbm, gtab_hbm,
             out_hbm, idxw, cntw, mskw, ww, widx, wrows, rows, outv,
             sem_a, sem_b, sem_w, sem_s, sem_o):
    wid = lax.axis_index("s") * 2 + lax.axis_index("c")

    pltpu.sync_copy(lg_hbm.at[wid], idxw)
    scp = [pltpu.async_copy(cnt_hbm.at[wid], cntw, sem_s),
           pltpu.async_copy(msk_hbm.at[wid], mskw, sem_s),
           pltpu.async_copy(words_hbm.at[wid], widx, sem_s)]

    sems = (sem_a, sem_b)

    def fire(c, par):
        s = sems[par]
        base = par * RPC
        return [
            pltpu.async_copy(gtab_hbm.at[idxw.at[c * C + p]],
                             rows.at[pl.ds(base + p * RPP, RPP)], s)
            for p in range(C)
        ]

    def wait_rows(c, par):
        s = sems[par]
        base = par * RPC
        for p in range(C):
            pltpu.make_async_copy(gtab_hbm.at[idxw.at[c * C + p]],
                                  rows.at[pl.ds(base + p * RPP, RPP)],
                                  s).wait()

    fire(0, 0)
    for cp in scp:
        cp.wait()
    wcp = pltpu.async_copy(wtab_hbm.at[widx], wrows, sem_w)

    lanes = lax.iota(jnp.int32, 16)

    def weight_body(q, carry):
        cs = [cntw[q, pl.ds(k * 16, 16)] for k in range(5)]
        s = cs[0] + cs[1] + cs[2] + cs[3] + cs[4]
        for sh in (1, 2, 4, 8):
            s = s + s.at[jnp.bitwise_xor(lanes, sh)].get(
                mode="promise_in_bounds")
        rs = 4.0 / s
        for k in range(5):
            m = mskw[q, pl.ds(k * 16, 16)]
            ww[q, pl.ds(k * 16, 16)] = cs[k] * rs * (1.0 - m)
        return carry

    lax.fori_loop(0, PPW, weight_body, 0)
    wcp.wait()

    def out_row0(c_first):
        return pl.multiple_of(wid * PPW + c_first * C, OB)

    def compute_chunk(c, par, ob):
        for p in range(C):
            qg = c * C + p
            po = (qg % OB)

            def layer_body(l, carry2, _p=p, _qg=qg, _po=po):
                b = l * GAZ_NUM
                wv0 = ww[_qg, pl.ds(b, 16)]
                wv1 = ww[_qg, pl.ds(b + 4, 16)]
                rbase = par * RPC + _p * RPP + b
                acc = [jnp.zeros((16,), jnp.float32) for _ in range(4)]
                for g in range(GAZ_NUM):
                    ws = wv0[g] if g < 16 else wv1[g - 4]
                    for v in range(4):
                        acc[v] = acc[v] + ws * rows[rbase + g,
                                                    pl.ds(v * 16, 16)]
                for v in range(4):
                    outv[ob, _po, pl.ds(WORD_DIM + l * 64 + v * 16, 16)] = \
                        acc[v]
                return carry2

            lax.fori_loop(0, NLAYER, layer_body, 0)
            for v in range(WORD_DIM // 16):
                outv[ob, po, pl.ds(v * 16, 16)] = wrows[qg, pl.ds(v * 16, 16)]

    def pair_body(t, carry):
        for b in range(2):
            c = 2 * t + b
            ob = (c // 4) % 2
            if b == 0:
                fire(c + 1, 1 - b)
            else:
                @pl.when(t < CPW // 2 - 1)
                def _():
                    fire(c + 1, 1 - b)
            wait_rows(c, b)
            if b == 0:
                @pl.when((t % 2 == 0) & (t >= 4))
                def _():
                    pltpu.make_async_copy(
                        outv.at[ob], out_hbm.at[pl.ds(out_row0(c - 8), OB)],
                        sem_o).wait()
            compute_chunk(c, b, ob)
            if b == 1:
                @pl.when(t % 2 == 1)
                def _():
                    pltpu.async_copy(
                        outv.at[ob], out_hbm.at[pl.ds(out_row0(c - 3), OB)],
                        sem_o)
        return carry

    lax.fori_loop(0, CPW // 2, pair_body, 0)
    for ob in range(2):
        pltpu.make_async_copy(
            outv.at[ob], out_hbm.at[pl.ds(wid * PPW, OB)], sem_o).wait()


@jax.jit
def kernel(words, layer_gazs, gaz_count, gaz_mask, word_table, gaz_table):
    lg = layer_gazs.reshape(NW, PPW, RPP).astype(jnp.int32)
    cnt = gaz_count.reshape(NW, PPW, RPP)
    msk = gaz_mask.reshape(NW, PPW, RPP).astype(jnp.float32)
    wds = words.reshape(NW, PPW).astype(jnp.int32)
    gtab = jnp.pad(gaz_table, ((0, 0), (0, PAIR_DIM - GAZ_DIM)))

    mesh = plsc.VectorSubcoreMesh(core_axis_name="c", subcore_axis_name="s")
    f = functools.partial(
        pl.kernel,
        out_type=jax.ShapeDtypeStruct((SEQ, OUT_DIM), jnp.float32),
        mesh=mesh,
        compiler_params=pltpu.CompilerParams(use_tc_tiling_on_sc=True),
        scratch_types=[
            pltpu.VMEM((PPW, RPP), jnp.int32),
            pltpu.VMEM((PPW, RPP), jnp.float32),
            pltpu.VMEM((PPW, RPP), jnp.float32),
            pltpu.VMEM((PPW, RPP), jnp.float32),
            pltpu.VMEM((PPW,), jnp.int32),
            pltpu.VMEM((PPW, WORD_DIM), jnp.float32),
            pltpu.VMEM((2 * RPC, PAIR_DIM), jnp.float32),
            pltpu.VMEM((2, OB, OUT_DIM), jnp.float32),
            pltpu.SemaphoreType.DMA,
            pltpu.SemaphoreType.DMA,
            pltpu.SemaphoreType.DMA,
            pltpu.SemaphoreType.DMA,
            pltpu.SemaphoreType.DMA,
        ],
    )(_sc_body)
    return f(lg, cnt, msk, wds, word_table, gtab)
